# baseline (speedup 1.0000x reference)
import functools

import jax
import jax.numpy as jnp
from jax import lax
from jax.experimental import pallas as pl
from jax.experimental.pallas import tpu as pltpu
from jax.experimental.pallas import tpu_sc as plsc

LANES = 16
EMBED = 64
BATCH = 16384
NUM_CORES = 2
NUM_SUBCORES = 16
NW = NUM_CORES * NUM_SUBCORES
B_PER_W = BATCH // NW
CHUNK = 64
N_CHUNKS = B_PER_W // CHUNK
GROUPS = CHUNK // LANES


def _mf_body(u_hbm, v_hbm, w_hbm, h_hbm, out_hbm,
             u_vm, v_vm, urows, vrows, out_v, *sems):
    wid = lax.axis_index("s") * NUM_CORES + lax.axis_index("c")
    base = wid * B_PER_W

    pltpu.sync_copy(u_hbm.at[pl.ds(base, B_PER_W)], u_vm)
    pltpu.sync_copy(v_hbm.at[pl.ds(base, B_PER_W)], v_vm)

    copies = []
    for ch in range(N_CHUNKS):
        sl = pl.ds(ch * CHUNK, CHUNK)
        cw = pltpu.async_copy(w_hbm.at[u_vm.at[sl]], urows.at[sl], sems[ch])
        chh = pltpu.async_copy(h_hbm.at[v_vm.at[sl]], vrows.at[sl], sems[ch])
        copies.append((cw, chh))

    lanes = lax.iota(jnp.int32, LANES)

    for ch in range(N_CHUNKS):
        cw, chh = copies[ch]
        cw.wait()
        chh.wait()

        def group(g, carry, ch=ch):
            off = ch * CHUNK + g * LANES
            rows = off + lanes
            acc0 = jnp.zeros((LANES,), jnp.float32)
            acc1 = jnp.zeros((LANES,), jnp.float32)
            for k in range(EMBED):
                kv = jnp.full((LANES,), k, jnp.int32)
                gu = plsc.load_gather(urows, [rows, kv])
                gv = plsc.load_gather(vrows, [rows, kv])
                if k % 2 == 0:
                    acc0 = acc0 + gu * gv
                else:
                    acc1 = acc1 + gu * gv
            out_v[pl.ds(off, LANES)] = acc0 + acc1
            return carry

        lax.fori_loop(0, GROUPS, group, 0)

    pltpu.sync_copy(out_v, out_hbm.at[pl.ds(base, B_PER_W)])


@jax.jit
def _mf_sc(u, v, W, H):
    mesh = plsc.VectorSubcoreMesh(core_axis_name="c", subcore_axis_name="s")
    return pl.kernel(
        _mf_body,
        mesh=mesh,
        compiler_params=pltpu.CompilerParams(
            needs_layout_passes=False, use_tc_tiling_on_sc=False),
        out_type=jax.ShapeDtypeStruct((BATCH,), jnp.float32),
        scratch_types=[
            pltpu.VMEM((B_PER_W,), jnp.int32),
            pltpu.VMEM((B_PER_W,), jnp.int32),
            pltpu.VMEM((B_PER_W, EMBED), jnp.float32),
            pltpu.VMEM((B_PER_W, EMBED), jnp.float32),
            pltpu.VMEM((B_PER_W,), jnp.float32),
        ] + [pltpu.SemaphoreType.DMA] * N_CHUNKS,
    )(u, v, W, H)


def _split_body(x_ref, u_ref, v_ref):
    blk = x_ref[...]
    u_ref[...] = blk[:, 0]
    v_ref[...] = blk[:, 1]


def _split_tc(x):
    return pl.pallas_call(
        _split_body,
        out_shape=[
            jax.ShapeDtypeStruct((BATCH,), jnp.int32),
            jax.ShapeDtypeStruct((BATCH,), jnp.int32),
        ],
    )(x)


def kernel(x, W, H):
    u, v = _split_tc(x)
    return _mf_sc(u, v, W, H)
